# baseline (speedup 1.0000x reference)
import math

import numpy as np
import jax
import jax.numpy as jnp
from jax.experimental import pallas as pl
from jax.experimental.pallas import tpu as pltpu

L = 2048
H = 16
E = 64
SAMPLE_K = 40
N_TOP = 40
SCALE = 1.0 / math.sqrt(E)
KT = 512
BT = 256


def _threefry2x32(k0, k1, x0, x1):
    def rotl(x, r):
        return ((x << np.uint32(r)) | (x >> np.uint32(32 - r))).astype(np.uint32)

    R = [13, 15, 26, 6, 17, 29, 16, 24]
    ks0, ks1 = np.uint32(k0), np.uint32(k1)
    ks2 = np.uint32(ks0 ^ ks1 ^ np.uint32(0x1BD11BDA))
    x0 = (x0 + ks0).astype(np.uint32)
    x1 = (x1 + ks1).astype(np.uint32)
    inject = [(ks1, ks2), (ks2, ks0), (ks0, ks1), (ks1, ks2), (ks2, ks0)]
    for g in range(5):
        for r in (R[0:4] if g % 2 == 0 else R[4:8]):
            x0 = (x0 + x1).astype(np.uint32)
            x1 = (rotl(x1, r) ^ x0).astype(np.uint32)
        a, b = inject[g]
        x0 = (x0 + a).astype(np.uint32)
        x1 = (x1 + b + np.uint32(g + 1)).astype(np.uint32)
    return x0, x1


def _sample_counts_t() -> np.ndarray:
    a, b = _threefry2x32(0, 42, np.zeros(2, np.uint32),
                         np.arange(2, dtype=np.uint32))
    k2 = (a[1], b[1])
    i = np.arange(L * SAMPLE_K, dtype=np.uint64)
    hi = (i >> np.uint64(32)).astype(np.uint32)
    lo = (i & np.uint64(0xFFFFFFFF)).astype(np.uint32)
    y0, y1 = _threefry2x32(k2[0], k2[1], hi, lo)
    idx = ((y0 ^ y1) % np.uint32(L)).astype(np.int32).reshape(L, SAMPLE_K)
    cnt = np.zeros((L, L), dtype=np.int8)
    np.add.at(cnt, (idx, np.arange(L)[:, None]), 1)
    return cnt


_COUNTS_T = _sample_counts_t()


def _split(x):
    hi = x.astype(jnp.bfloat16)
    lo = (x - hi.astype(jnp.float32)).astype(jnp.bfloat16)
    return hi, lo


def _mm(a, b, dims):
    return jax.lax.dot_general(a, b, (dims, ((), ())),
                               preferred_element_type=jnp.float32)


def _mm3(a, b, dims):
    ah, al = _split(a)
    bh, bl = _split(b)
    return _mm(ah, bh, dims) + (_mm(ah, bl, dims) + _mm(al, bh, dims))


def _one_head(q, k, v, c_ref):
    qb = q.astype(jnp.bfloat16)
    kb = k.astype(jnp.bfloat16)
    run_max = jnp.full((1, L), -jnp.inf, dtype=jnp.float32)
    run_sum = jnp.zeros((1, L), dtype=jnp.float32)
    for t in range(L // KT):
        ktile = kb[t * KT:(t + 1) * KT, :]
        st = _mm(ktile, qb, ((1,), (1,)))
        cf = c_ref[t * KT:(t + 1) * KT, :].astype(jnp.float32)
        run_sum = run_sum + jnp.sum(st * cf, axis=0, keepdims=True)
        masked = jnp.where(cf > 0.0, st, -jnp.inf)
        run_max = jnp.maximum(run_max, jnp.max(masked, axis=0, keepdims=True))
    sparsity = run_max - run_sum * (1.0 / SAMPLE_K)

    iota_row = jax.lax.broadcasted_iota(jnp.int32, (1, L), 1)
    iota_col = jax.lax.broadcasted_iota(jnp.int32, (L, 1), 0)
    iota16c = jax.lax.broadcasted_iota(jnp.int32, (16, 1), 0)
    bias = jnp.int32(-2 ** 31)

    bits = jax.lax.bitcast_convert_type(sparsity, jnp.int32)
    u = jnp.where(bits < 0, bits ^ jnp.int32(0x7FFFFFFF), bits)

    tp = jnp.zeros((1, 1), jnp.int32)
    for b in range(28, -1, -4):
        th_biased = tp | (iota16c << b)
        cmp = u >= (th_biased ^ bias)
        cnt = jnp.sum(jnp.where(cmp, 1, 0), axis=1, keepdims=True)
        nib = jnp.sum(jnp.where(cnt >= N_TOP, 1, 0),
                      axis=0, keepdims=True) - 1
        tp = tp | (nib << b)
    thr = tp ^ bias

    gt = u > thr
    eq = u == thr
    need = N_TOP - jnp.sum(jnp.where(gt, 1, 0), keepdims=True)

    ip = jnp.zeros((1, 1), jnp.int32)
    for b in (8, 4, 0):
        th_hi = ip + ((iota16c + 1) << b) - 1
        cmp = eq & (iota_row <= th_hi)
        cnt = jnp.sum(jnp.where(cmp, 1, 0), axis=1, keepdims=True)
        nib = jnp.sum(jnp.where(cnt < need, 1, 0), axis=0, keepdims=True)
        ip = ip + (nib << b)
    memb = gt | (eq & (iota_row <= ip))

    memb_f = jnp.where(memb, 1.0, 0.0)
    m2 = jnp.concatenate(
        [memb_f[:, i * 128:(i + 1) * 128] for i in range(L // 128)], axis=0)
    m2b = m2.astype(jnp.bfloat16)
    ur = jax.lax.broadcasted_iota(jnp.int32, (128, 128), 0)
    uc = jax.lax.broadcasted_iota(jnp.int32, (128, 128), 1)
    triu_b = (ur <= uc).astype(jnp.bfloat16)
    within = _mm(m2b, triu_b, ((1,), (0,)))
    rowtot_b = within[:, 127:128].astype(jnp.bfloat16)
    r16 = jax.lax.broadcasted_iota(jnp.int32, (16, 16), 0)
    c16 = jax.lax.broadcasted_iota(jnp.int32, (16, 16), 1)
    t16_b = (r16 > c16).astype(jnp.bfloat16)
    offs = _mm(t16_b, rowtot_b, ((1,), (0,)))
    rank = within + offs - m2
    rank_row = jnp.concatenate(
        [rank[s:s + 1, :] for s in range(L // 128)], axis=1)

    col40f = jax.lax.broadcasted_iota(
        jnp.int32, (N_TOP, 1), 0).astype(jnp.float32)
    p_t = jnp.where((rank_row == col40f) & memb, 1.0, 0.0)
    pb = p_t.astype(jnp.bfloat16)

    iota_rowf = iota_row.astype(jnp.float32)
    ti_col = jnp.sum(p_t * iota_rowf, axis=1,
                     keepdims=True).astype(jnp.int32)

    qh, ql = _split(q)
    q_top = _mm(pb, qh, ((1,), (0,))) + _mm(pb, ql, ((1,), (0,)))
    scores = _mm3(q_top, k, ((1,), (1,))) * SCALE
    key_pos = jax.lax.broadcasted_iota(jnp.int32, (N_TOP, L), 1)
    scores = jnp.where(key_pos > ti_col, -jnp.inf, scores)
    smax = jnp.max(scores, axis=1, keepdims=True)
    ex = jnp.exp(scores - smax)
    attn = ex / jnp.sum(ex, axis=1, keepdims=True)
    updates = _mm3(attn, v, ((1,), (0,)))

    ri = jax.lax.broadcasted_iota(jnp.int32, (BT, BT), 0)
    ci = jax.lax.broadcasted_iota(jnp.int32, (BT, BT), 1)
    trib = (ri >= ci).astype(jnp.bfloat16)
    vh, vl = _split(v)
    prefix = jnp.zeros((1, E), jnp.float32)
    blocks = []
    for b in range(L // BT):
        sl = slice(b * BT, (b + 1) * BT)
        cb = (_mm(trib, vh[sl], ((1,), (0,))) +
              _mm(trib, vl[sl], ((1,), (0,))) + prefix)
        blocks.append(cb)
        prefix = cb[BT - 1:BT, :]
    ctx = jnp.concatenate(blocks, axis=0)

    uh, ul = _split(updates)
    scattered = _mm(pb, uh, ((0,), (0,))) + _mm(pb, ul, ((0,), (0,)))
    ones40_b = jnp.ones((N_TOP, 1), jnp.bfloat16)
    is_top = _mm(pb, ones40_b, ((0,), (0,))) > 0.0
    return jnp.where(is_top, scattered, ctx)


def _body(q_ref, k_ref, v_ref, c_ref, o_ref):
    for i in range(2):
        sl = slice(i * E, (i + 1) * E)
        o_ref[:, sl] = _one_head(q_ref[:, sl], k_ref[:, sl], v_ref[:, sl],
                                 c_ref)


def kernel(queries, keys, values):
    B, Lq, Hn, En = queries.shape
    q2 = queries.reshape(L, H * E)
    k2 = keys.reshape(L, H * E)
    v2 = values.reshape(L, H * E)
    counts_t = jnp.asarray(_COUNTS_T)

    spec = pl.BlockSpec((L, 2 * E), lambda h: (0, h))
    spec_c = pl.BlockSpec((L, L), lambda h: (0, 0))
    out = pl.pallas_call(
        _body,
        grid=(H // 2,),
        in_specs=[spec, spec, spec, spec_c],
        out_specs=spec,
        out_shape=jax.ShapeDtypeStruct((L, H * E), jnp.float32),
        compiler_params=pltpu.CompilerParams(
            dimension_semantics=("arbitrary",)),
    )(q2, k2, v2, counts_t)
    return out.reshape(B, Lq, Hn, En)


# speedup vs baseline: 1.0761x; 1.0761x over previous
import math

import numpy as np
import jax
import jax.numpy as jnp
from jax.experimental import pallas as pl
from jax.experimental.pallas import tpu as pltpu

L = 2048
H = 16
E = 64
SAMPLE_K = 40
N_TOP = 40
SCALE = 1.0 / math.sqrt(E)
KT = 512
BT = 256


def _threefry2x32(k0, k1, x0, x1):
    def rotl(x, r):
        return ((x << np.uint32(r)) | (x >> np.uint32(32 - r))).astype(np.uint32)

    R = [13, 15, 26, 6, 17, 29, 16, 24]
    ks0, ks1 = np.uint32(k0), np.uint32(k1)
    ks2 = np.uint32(ks0 ^ ks1 ^ np.uint32(0x1BD11BDA))
    x0 = (x0 + ks0).astype(np.uint32)
    x1 = (x1 + ks1).astype(np.uint32)
    inject = [(ks1, ks2), (ks2, ks0), (ks0, ks1), (ks1, ks2), (ks2, ks0)]
    for g in range(5):
        for r in (R[0:4] if g % 2 == 0 else R[4:8]):
            x0 = (x0 + x1).astype(np.uint32)
            x1 = (rotl(x1, r) ^ x0).astype(np.uint32)
        a, b = inject[g]
        x0 = (x0 + a).astype(np.uint32)
        x1 = (x1 + b + np.uint32(g + 1)).astype(np.uint32)
    return x0, x1


def _sample_counts_t() -> np.ndarray:
    a, b = _threefry2x32(0, 42, np.zeros(2, np.uint32),
                         np.arange(2, dtype=np.uint32))
    k2 = (a[1], b[1])
    i = np.arange(L * SAMPLE_K, dtype=np.uint64)
    hi = (i >> np.uint64(32)).astype(np.uint32)
    lo = (i & np.uint64(0xFFFFFFFF)).astype(np.uint32)
    y0, y1 = _threefry2x32(k2[0], k2[1], hi, lo)
    idx = ((y0 ^ y1) % np.uint32(L)).astype(np.int32).reshape(L, SAMPLE_K)
    cnt = np.zeros((L, L), dtype=np.int8)
    np.add.at(cnt, (idx, np.arange(L)[:, None]), 1)
    return cnt


_COUNTS_T = _sample_counts_t()
_MASK_NEG = np.where(_COUNTS_T > 0, np.float32(0.0),
                     np.float32(-1e30)).astype(np.float32)


def _split(x):
    hi = x.astype(jnp.bfloat16)
    lo = (x - hi.astype(jnp.float32)).astype(jnp.bfloat16)
    return hi, lo


def _mm(a, b, dims):
    return jax.lax.dot_general(a, b, (dims, ((), ())),
                               preferred_element_type=jnp.float32)


def _mm3(a, b, dims):
    ah, al = _split(a)
    bh, bl = _split(b)
    return _mm(ah, bh, dims) + (_mm(ah, bl, dims) + _mm(al, bh, dims))


def _one_head(q, k, v, c_ref, m_ref):
    qb = q.astype(jnp.bfloat16)
    kb = k.astype(jnp.bfloat16)
    run_max = jnp.full((1, L), -jnp.inf, dtype=jnp.float32)
    run_sum = jnp.zeros((1, L), dtype=jnp.float32)
    for t in range(L // KT):
        ktile = kb[t * KT:(t + 1) * KT, :]
        st = _mm(ktile, qb, ((1,), (1,)))
        cf = c_ref[t * KT:(t + 1) * KT, :].astype(jnp.float32)
        run_sum = run_sum + jnp.sum(st * cf, axis=0, keepdims=True)
        masked = st + m_ref[t * KT:(t + 1) * KT, :]
        run_max = jnp.maximum(run_max, jnp.max(masked, axis=0, keepdims=True))
    sparsity = run_max - run_sum * (1.0 / SAMPLE_K)

    iota_row = jax.lax.broadcasted_iota(jnp.int32, (1, L), 1)
    iota_col = jax.lax.broadcasted_iota(jnp.int32, (L, 1), 0)
    iota16c = jax.lax.broadcasted_iota(jnp.int32, (16, 1), 0)
    bias = jnp.int32(-2 ** 31)

    bits = jax.lax.bitcast_convert_type(sparsity, jnp.int32)
    u = jnp.where(bits < 0, bits ^ jnp.int32(0x7FFFFFFF), bits)

    tp = jnp.zeros((1, 1), jnp.int32)
    for b in range(28, -1, -4):
        th_biased = tp | (iota16c << b)
        cmp = u >= (th_biased ^ bias)
        cnt = jnp.sum(jnp.where(cmp, 1, 0), axis=1, keepdims=True)
        nib = jnp.sum(jnp.where(cnt >= N_TOP, 1, 0),
                      axis=0, keepdims=True) - 1
        tp = tp | (nib << b)
    thr = tp ^ bias

    gt = u > thr
    eq = u == thr
    need = N_TOP - jnp.sum(jnp.where(gt, 1, 0), keepdims=True)

    ip = jnp.zeros((1, 1), jnp.int32)
    for b in (8, 4, 0):
        th_hi = ip + ((iota16c + 1) << b) - 1
        cmp = eq & (iota_row <= th_hi)
        cnt = jnp.sum(jnp.where(cmp, 1, 0), axis=1, keepdims=True)
        nib = jnp.sum(jnp.where(cnt < need, 1, 0), axis=0, keepdims=True)
        ip = ip + (nib << b)
    memb = gt | (eq & (iota_row <= ip))

    memb_f = jnp.where(memb, 1.0, 0.0)
    m2 = jnp.concatenate(
        [memb_f[:, i * 128:(i + 1) * 128] for i in range(L // 128)], axis=0)
    m2b = m2.astype(jnp.bfloat16)
    ur = jax.lax.broadcasted_iota(jnp.int32, (128, 128), 0)
    uc = jax.lax.broadcasted_iota(jnp.int32, (128, 128), 1)
    triu_b = (ur <= uc).astype(jnp.bfloat16)
    within = _mm(m2b, triu_b, ((1,), (0,)))
    rowtot_b = within[:, 127:128].astype(jnp.bfloat16)
    r16 = jax.lax.broadcasted_iota(jnp.int32, (16, 16), 0)
    c16 = jax.lax.broadcasted_iota(jnp.int32, (16, 16), 1)
    t16_b = (r16 > c16).astype(jnp.bfloat16)
    offs = _mm(t16_b, rowtot_b, ((1,), (0,)))
    rank = within + offs - m2
    rank_row = jnp.concatenate(
        [rank[s:s + 1, :] for s in range(L // 128)], axis=1)

    col40f = jax.lax.broadcasted_iota(
        jnp.int32, (N_TOP, 1), 0).astype(jnp.float32)
    p_t = jnp.where((rank_row == col40f) & memb, 1.0, 0.0)
    pb = p_t.astype(jnp.bfloat16)

    iota_rowf = iota_row.astype(jnp.float32)
    ti_col = jnp.sum(p_t * iota_rowf, axis=1,
                     keepdims=True).astype(jnp.int32)

    q_top = _mm(pb, qb, ((1,), (0,)))
    scores = _mm(q_top.astype(jnp.bfloat16), kb, ((1,), (1,))) * SCALE
    key_pos = jax.lax.broadcasted_iota(jnp.int32, (N_TOP, L), 1)
    scores = jnp.where(key_pos > ti_col, -jnp.inf, scores)
    smax = jnp.max(scores, axis=1, keepdims=True)
    ex = jnp.exp(scores - smax)
    attn = ex / jnp.sum(ex, axis=1, keepdims=True)
    updates = _mm(attn.astype(jnp.bfloat16), v.astype(jnp.bfloat16),
                  ((1,), (0,)))

    ri = jax.lax.broadcasted_iota(jnp.int32, (BT, BT), 0)
    ci = jax.lax.broadcasted_iota(jnp.int32, (BT, BT), 1)
    trib = (ri >= ci).astype(jnp.bfloat16)
    vh, vl = _split(v)
    prefix = jnp.zeros((1, E), jnp.float32)
    blocks = []
    for b in range(L // BT):
        sl = slice(b * BT, (b + 1) * BT)
        cb = (_mm(trib, vh[sl], ((1,), (0,))) +
              _mm(trib, vl[sl], ((1,), (0,))) + prefix)
        blocks.append(cb)
        prefix = cb[BT - 1:BT, :]
    ctx = jnp.concatenate(blocks, axis=0)

    scattered = _mm(pb, updates.astype(jnp.bfloat16), ((0,), (0,)))
    ones40_b = jnp.ones((N_TOP, 1), jnp.bfloat16)
    is_top = _mm(pb, ones40_b, ((0,), (0,))) > 0.0
    return jnp.where(is_top, scattered, ctx)


def _body(q_ref, k_ref, v_ref, c_ref, m_ref, o_ref):
    for i in range(2):
        sl = slice(i * E, (i + 1) * E)
        o_ref[:, sl] = _one_head(q_ref[:, sl], k_ref[:, sl], v_ref[:, sl],
                                 c_ref, m_ref)


def kernel(queries, keys, values):
    B, Lq, Hn, En = queries.shape
    q2 = queries.reshape(L, H * E)
    k2 = keys.reshape(L, H * E)
    v2 = values.reshape(L, H * E)
    counts_t = jnp.asarray(_COUNTS_T)
    msk = jnp.asarray(_MASK_NEG)

    spec = pl.BlockSpec((L, 2 * E), lambda h: (0, h))
    spec_c = pl.BlockSpec((L, L), lambda h: (0, 0))
    out = pl.pallas_call(
        _body,
        grid=(H // 2,),
        in_specs=[spec, spec, spec, spec_c, spec_c],
        out_specs=spec,
        out_shape=jax.ShapeDtypeStruct((L, H * E), jnp.float32),
        compiler_params=pltpu.CompilerParams(
            dimension_semantics=("arbitrary",)),
    )(q2, k2, v2, counts_t, msk)
    return out.reshape(B, Lq, Hn, En)


# speedup vs baseline: 1.1187x; 1.0396x over previous
import math

import numpy as np
import jax
import jax.numpy as jnp
from jax.experimental import pallas as pl
from jax.experimental.pallas import tpu as pltpu

L = 2048
H = 16
E = 64
SAMPLE_K = 40
N_TOP = 40
SCALE = 1.0 / math.sqrt(E)
KT = 512
BT = 256


def _threefry2x32(k0, k1, x0, x1):
    def rotl(x, r):
        return ((x << np.uint32(r)) | (x >> np.uint32(32 - r))).astype(np.uint32)

    R = [13, 15, 26, 6, 17, 29, 16, 24]
    ks0, ks1 = np.uint32(k0), np.uint32(k1)
    ks2 = np.uint32(ks0 ^ ks1 ^ np.uint32(0x1BD11BDA))
    x0 = (x0 + ks0).astype(np.uint32)
    x1 = (x1 + ks1).astype(np.uint32)
    inject = [(ks1, ks2), (ks2, ks0), (ks0, ks1), (ks1, ks2), (ks2, ks0)]
    for g in range(5):
        for r in (R[0:4] if g % 2 == 0 else R[4:8]):
            x0 = (x0 + x1).astype(np.uint32)
            x1 = (rotl(x1, r) ^ x0).astype(np.uint32)
        a, b = inject[g]
        x0 = (x0 + a).astype(np.uint32)
        x1 = (x1 + b + np.uint32(g + 1)).astype(np.uint32)
    return x0, x1


def _sample_counts_t() -> np.ndarray:
    a, b = _threefry2x32(0, 42, np.zeros(2, np.uint32),
                         np.arange(2, dtype=np.uint32))
    k2 = (a[1], b[1])
    i = np.arange(L * SAMPLE_K, dtype=np.uint64)
    hi = (i >> np.uint64(32)).astype(np.uint32)
    lo = (i & np.uint64(0xFFFFFFFF)).astype(np.uint32)
    y0, y1 = _threefry2x32(k2[0], k2[1], hi, lo)
    idx = ((y0 ^ y1) % np.uint32(L)).astype(np.int32).reshape(L, SAMPLE_K)
    cnt = np.zeros((L, L), dtype=np.int8)
    np.add.at(cnt, (idx, np.arange(L)[:, None]), 1)
    return cnt


_COUNTS_T = _sample_counts_t()
_MASK_NEG = np.where(_COUNTS_T > 0, np.float32(0.0),
                     np.float32(-1e30)).astype(np.float32)


def _split(x):
    hi = x.astype(jnp.bfloat16)
    lo = (x - hi.astype(jnp.float32)).astype(jnp.bfloat16)
    return hi, lo


def _mm(a, b, dims):
    return jax.lax.dot_general(a, b, (dims, ((), ())),
                               preferred_element_type=jnp.float32)


def _mm3(a, b, dims):
    ah, al = _split(a)
    bh, bl = _split(b)
    return _mm(ah, bh, dims) + (_mm(ah, bl, dims) + _mm(al, bh, dims))


def _cumsum_block(v):
    ri = jax.lax.broadcasted_iota(jnp.int32, (BT, BT), 0)
    ci = jax.lax.broadcasted_iota(jnp.int32, (BT, BT), 1)
    trib = (ri >= ci).astype(jnp.bfloat16)
    vh, vl = _split(v)
    prefix = jnp.zeros((1, v.shape[1]), jnp.float32)
    blocks = []
    for b in range(L // BT):
        sl = slice(b * BT, (b + 1) * BT)
        cb = (_mm(trib, vh[sl], ((1,), (0,))) +
              _mm(trib, vl[sl], ((1,), (0,))) + prefix)
        blocks.append(cb)
        prefix = cb[BT - 1:BT, :]
    return jnp.concatenate(blocks, axis=0)


def _one_head(q, k, v, ctx, c_ref, m_ref):
    qb = q.astype(jnp.bfloat16)
    kb = k.astype(jnp.bfloat16)
    run_max = jnp.full((1, L), -jnp.inf, dtype=jnp.float32)
    run_sum = jnp.zeros((1, L), dtype=jnp.float32)
    for t in range(L // KT):
        ktile = kb[t * KT:(t + 1) * KT, :]
        st = _mm(ktile, qb, ((1,), (1,)))
        cf = c_ref[t * KT:(t + 1) * KT, :].astype(jnp.float32)
        run_sum = run_sum + jnp.sum(st * cf, axis=0, keepdims=True)
        masked = st + m_ref[t * KT:(t + 1) * KT, :]
        run_max = jnp.maximum(run_max, jnp.max(masked, axis=0, keepdims=True))
    sparsity = run_max - run_sum * (1.0 / SAMPLE_K)

    iota_row = jax.lax.broadcasted_iota(jnp.int32, (1, L), 1)
    iota_col = jax.lax.broadcasted_iota(jnp.int32, (L, 1), 0)
    iota16c = jax.lax.broadcasted_iota(jnp.int32, (16, 1), 0)
    bias = jnp.int32(-2 ** 31)

    bits = jax.lax.bitcast_convert_type(sparsity, jnp.int32)
    u = jnp.where(bits < 0, bits ^ jnp.int32(0x7FFFFFFF), bits)

    tp = jnp.zeros((1, 1), jnp.int32)
    for b in range(28, -1, -4):
        th_biased = tp | (iota16c << b)
        cmp = u >= (th_biased ^ bias)
        cnt = jnp.sum(jnp.where(cmp, 1, 0), axis=1, keepdims=True)
        nib = jnp.sum(jnp.where(cnt >= N_TOP, 1, 0),
                      axis=0, keepdims=True) - 1
        tp = tp | (nib << b)
    thr = tp ^ bias

    gt = u > thr
    eq = u == thr
    need = N_TOP - jnp.sum(jnp.where(gt, 1, 0), keepdims=True)

    ip = jnp.zeros((1, 1), jnp.int32)
    for b in (8, 4, 0):
        th_hi = ip + ((iota16c + 1) << b) - 1
        cmp = eq & (iota_row <= th_hi)
        cnt = jnp.sum(jnp.where(cmp, 1, 0), axis=1, keepdims=True)
        nib = jnp.sum(jnp.where(cnt < need, 1, 0), axis=0, keepdims=True)
        ip = ip + (nib << b)
    memb = gt | (eq & (iota_row <= ip))

    memb_f = jnp.where(memb, 1.0, 0.0)
    m2 = jnp.concatenate(
        [memb_f[:, i * 128:(i + 1) * 128] for i in range(L // 128)], axis=0)
    m2b = m2.astype(jnp.bfloat16)
    ur = jax.lax.broadcasted_iota(jnp.int32, (128, 128), 0)
    uc = jax.lax.broadcasted_iota(jnp.int32, (128, 128), 1)
    triu_b = (ur <= uc).astype(jnp.bfloat16)
    within = _mm(m2b, triu_b, ((1,), (0,)))
    rowtot_b = within[:, 127:128].astype(jnp.bfloat16)
    r16 = jax.lax.broadcasted_iota(jnp.int32, (16, 16), 0)
    c16 = jax.lax.broadcasted_iota(jnp.int32, (16, 16), 1)
    t16_b = (r16 > c16).astype(jnp.bfloat16)
    offs = _mm(t16_b, rowtot_b, ((1,), (0,)))
    rank = within + offs - m2
    rank_row = jnp.concatenate(
        [rank[s:s + 1, :] for s in range(L // 128)], axis=1)

    col40f = jax.lax.broadcasted_iota(
        jnp.int32, (N_TOP, 1), 0).astype(jnp.float32)
    p_t = jnp.where((rank_row == col40f) & memb, 1.0, 0.0)
    pb = p_t.astype(jnp.bfloat16)

    iota_rowf = iota_row.astype(jnp.float32)
    ti_col = jnp.sum(p_t * iota_rowf, axis=1,
                     keepdims=True).astype(jnp.int32)

    q_top = _mm(pb, qb, ((1,), (0,)))
    scores = _mm(q_top.astype(jnp.bfloat16), kb, ((1,), (1,))) * SCALE
    key_pos = jax.lax.broadcasted_iota(jnp.int32, (N_TOP, L), 1)
    scores = jnp.where(key_pos > ti_col, -jnp.inf, scores)
    smax = jnp.max(scores, axis=1, keepdims=True)
    ex = jnp.exp(scores - smax)
    attn = ex / jnp.sum(ex, axis=1, keepdims=True)
    updates = _mm(attn.astype(jnp.bfloat16), v.astype(jnp.bfloat16),
                  ((1,), (0,)))

    scattered = _mm(pb, updates.astype(jnp.bfloat16), ((0,), (0,)))
    ones40_b = jnp.ones((N_TOP, 1), jnp.bfloat16)
    is_top = _mm(pb, ones40_b, ((0,), (0,))) > 0.0
    return jnp.where(is_top, scattered, ctx)


def _body(q_ref, k_ref, v_ref, c_ref, m_ref, o_ref):
    ctx2 = _cumsum_block(v_ref[...])
    for i in range(2):
        sl = slice(i * E, (i + 1) * E)
        o_ref[:, sl] = _one_head(q_ref[:, sl], k_ref[:, sl], v_ref[:, sl],
                                 ctx2[:, sl], c_ref, m_ref)


def kernel(queries, keys, values):
    B, Lq, Hn, En = queries.shape
    q2 = queries.reshape(L, H * E)
    k2 = keys.reshape(L, H * E)
    v2 = values.reshape(L, H * E)
    counts_t = jnp.asarray(_COUNTS_T)
    msk = jnp.asarray(_MASK_NEG)

    spec = pl.BlockSpec((L, 2 * E), lambda h: (0, h))
    spec_c = pl.BlockSpec((L, L), lambda h: (0, 0))
    out = pl.pallas_call(
        _body,
        grid=(H // 2,),
        in_specs=[spec, spec, spec, spec_c, spec_c],
        out_specs=spec,
        out_shape=jax.ShapeDtypeStruct((L, H * E), jnp.float32),
        compiler_params=pltpu.CompilerParams(
            dimension_semantics=("parallel",)),
    )(q2, k2, v2, counts_t, msk)
    return out.reshape(B, Lq, Hn, En)


# speedup vs baseline: 1.1296x; 1.0097x over previous
import math

import numpy as np
import jax
import jax.numpy as jnp
from jax.experimental import pallas as pl
from jax.experimental.pallas import tpu as pltpu

L = 2048
H = 16
E = 64
SAMPLE_K = 40
N_TOP = 40
SCALE = 1.0 / math.sqrt(E)
KT = 1024
BT = 256


def _threefry2x32(k0, k1, x0, x1):
    def rotl(x, r):
        return ((x << np.uint32(r)) | (x >> np.uint32(32 - r))).astype(np.uint32)

    R = [13, 15, 26, 6, 17, 29, 16, 24]
    ks0, ks1 = np.uint32(k0), np.uint32(k1)
    ks2 = np.uint32(ks0 ^ ks1 ^ np.uint32(0x1BD11BDA))
    x0 = (x0 + ks0).astype(np.uint32)
    x1 = (x1 + ks1).astype(np.uint32)
    inject = [(ks1, ks2), (ks2, ks0), (ks0, ks1), (ks1, ks2), (ks2, ks0)]
    for g in range(5):
        for r in (R[0:4] if g % 2 == 0 else R[4:8]):
            x0 = (x0 + x1).astype(np.uint32)
            x1 = (rotl(x1, r) ^ x0).astype(np.uint32)
        a, b = inject[g]
        x0 = (x0 + a).astype(np.uint32)
        x1 = (x1 + b + np.uint32(g + 1)).astype(np.uint32)
    return x0, x1


def _sample_counts_t() -> np.ndarray:
    a, b = _threefry2x32(0, 42, np.zeros(2, np.uint32),
                         np.arange(2, dtype=np.uint32))
    k2 = (a[1], b[1])
    i = np.arange(L * SAMPLE_K, dtype=np.uint64)
    hi = (i >> np.uint64(32)).astype(np.uint32)
    lo = (i & np.uint64(0xFFFFFFFF)).astype(np.uint32)
    y0, y1 = _threefry2x32(k2[0], k2[1], hi, lo)
    idx = ((y0 ^ y1) % np.uint32(L)).astype(np.int32).reshape(L, SAMPLE_K)
    cnt = np.zeros((L, L), dtype=np.int8)
    np.add.at(cnt, (idx, np.arange(L)[:, None]), 1)
    return cnt


_COUNTS_T = _sample_counts_t()
_MASK_NEG = np.where(_COUNTS_T > 0, np.float32(0.0),
                     np.float32(-1e30)).astype(np.float32)


def _split(x):
    hi = x.astype(jnp.bfloat16)
    lo = (x - hi.astype(jnp.float32)).astype(jnp.bfloat16)
    return hi, lo


def _mm(a, b, dims):
    return jax.lax.dot_general(a, b, (dims, ((), ())),
                               preferred_element_type=jnp.float32)


def _mm3(a, b, dims):
    ah, al = _split(a)
    bh, bl = _split(b)
    return _mm(ah, bh, dims) + (_mm(ah, bl, dims) + _mm(al, bh, dims))


def _cumsum_block(v):
    ri = jax.lax.broadcasted_iota(jnp.int32, (BT, BT), 0)
    ci = jax.lax.broadcasted_iota(jnp.int32, (BT, BT), 1)
    trib = (ri >= ci).astype(jnp.bfloat16)
    vh, vl = _split(v)
    prefix = jnp.zeros((1, v.shape[1]), jnp.float32)
    blocks = []
    for b in range(L // BT):
        sl = slice(b * BT, (b + 1) * BT)
        cb = (_mm(trib, vh[sl], ((1,), (0,))) +
              _mm(trib, vl[sl], ((1,), (0,))) + prefix)
        blocks.append(cb)
        prefix = cb[BT - 1:BT, :]
    return jnp.concatenate(blocks, axis=0)


def _one_head(q, k, v, ctx, c_ref, m_ref):
    qb = q.astype(jnp.bfloat16)
    kb = k.astype(jnp.bfloat16)
    run_max = jnp.full((1, L), -jnp.inf, dtype=jnp.float32)
    run_sum = jnp.zeros((1, L), dtype=jnp.float32)
    for t in range(L // KT):
        ktile = kb[t * KT:(t + 1) * KT, :]
        st = _mm(ktile, qb, ((1,), (1,)))
        cf = c_ref[t * KT:(t + 1) * KT, :].astype(jnp.float32)
        run_sum = run_sum + jnp.sum(st * cf, axis=0, keepdims=True)
        masked = st + m_ref[t * KT:(t + 1) * KT, :]
        run_max = jnp.maximum(run_max, jnp.max(masked, axis=0, keepdims=True))
    sparsity = run_max - run_sum * (1.0 / SAMPLE_K)

    iota_row = jax.lax.broadcasted_iota(jnp.int32, (1, L), 1)
    iota_col = jax.lax.broadcasted_iota(jnp.int32, (L, 1), 0)
    iota16c = jax.lax.broadcasted_iota(jnp.int32, (16, 1), 0)
    bias = jnp.int32(-2 ** 31)

    bits = jax.lax.bitcast_convert_type(sparsity, jnp.int32)
    u = jnp.where(bits < 0, bits ^ jnp.int32(0x7FFFFFFF), bits)

    tp = jnp.zeros((1, 1), jnp.int32)
    for b in range(28, -1, -4):
        th_biased = tp | (iota16c << b)
        cmp = u >= (th_biased ^ bias)
        cnt = jnp.sum(jnp.where(cmp, 1, 0), axis=1, keepdims=True)
        nib = jnp.sum(jnp.where(cnt >= N_TOP, 1, 0),
                      axis=0, keepdims=True) - 1
        tp = tp | (nib << b)
    thr = tp ^ bias

    gt = u > thr
    eq = u == thr
    need = N_TOP - jnp.sum(jnp.where(gt, 1, 0), keepdims=True)

    ip = jnp.zeros((1, 1), jnp.int32)
    for b in (8, 4, 0):
        th_hi = ip + ((iota16c + 1) << b) - 1
        cmp = eq & (iota_row <= th_hi)
        cnt = jnp.sum(jnp.where(cmp, 1, 0), axis=1, keepdims=True)
        nib = jnp.sum(jnp.where(cnt < need, 1, 0), axis=0, keepdims=True)
        ip = ip + (nib << b)
    memb = gt | (eq & (iota_row <= ip))

    memb_f = jnp.where(memb, 1.0, 0.0)
    m2 = jnp.concatenate(
        [memb_f[:, i * 128:(i + 1) * 128] for i in range(L // 128)], axis=0)
    m2b = m2.astype(jnp.bfloat16)
    ur = jax.lax.broadcasted_iota(jnp.int32, (128, 128), 0)
    uc = jax.lax.broadcasted_iota(jnp.int32, (128, 128), 1)
    triu_b = (ur <= uc).astype(jnp.bfloat16)
    within = _mm(m2b, triu_b, ((1,), (0,)))
    rowtot_b = within[:, 127:128].astype(jnp.bfloat16)
    r16 = jax.lax.broadcasted_iota(jnp.int32, (16, 16), 0)
    c16 = jax.lax.broadcasted_iota(jnp.int32, (16, 16), 1)
    t16_b = (r16 > c16).astype(jnp.bfloat16)
    offs = _mm(t16_b, rowtot_b, ((1,), (0,)))
    rank = within + offs - m2
    rank_row = jnp.concatenate(
        [rank[s:s + 1, :] for s in range(L // 128)], axis=1)

    col40f = jax.lax.broadcasted_iota(
        jnp.int32, (N_TOP, 1), 0).astype(jnp.float32)
    p_t = jnp.where((rank_row == col40f) & memb, 1.0, 0.0)
    pb = p_t.astype(jnp.bfloat16)

    iota_rowf = iota_row.astype(jnp.float32)
    ti_col = jnp.sum(p_t * iota_rowf, axis=1,
                     keepdims=True).astype(jnp.int32)

    q_top = _mm(pb, qb, ((1,), (0,)))
    scores = _mm(q_top.astype(jnp.bfloat16), kb, ((1,), (1,))) * SCALE
    key_pos = jax.lax.broadcasted_iota(jnp.int32, (N_TOP, L), 1)
    scores = jnp.where(key_pos > ti_col, -jnp.inf, scores)
    smax = jnp.max(scores, axis=1, keepdims=True)
    ex = jnp.exp(scores - smax)
    attn = ex / jnp.sum(ex, axis=1, keepdims=True)
    updates = _mm(attn.astype(jnp.bfloat16), v.astype(jnp.bfloat16),
                  ((1,), (0,)))

    scattered = _mm(pb, updates.astype(jnp.bfloat16), ((0,), (0,)))
    ones40_b = jnp.ones((N_TOP, 1), jnp.bfloat16)
    is_top = _mm(pb, ones40_b, ((0,), (0,))) > 0.0
    return jnp.where(is_top, scattered, ctx)


def _body(q_ref, k_ref, v_ref, c_ref, m_ref, o_ref):
    ctx2 = _cumsum_block(v_ref[...])
    for i in range(2):
        sl = slice(i * E, (i + 1) * E)
        o_ref[:, sl] = _one_head(q_ref[:, sl], k_ref[:, sl], v_ref[:, sl],
                                 ctx2[:, sl], c_ref, m_ref)


def kernel(queries, keys, values):
    B, Lq, Hn, En = queries.shape
    q2 = queries.reshape(L, H * E)
    k2 = keys.reshape(L, H * E)
    v2 = values.reshape(L, H * E)
    counts_t = jnp.asarray(_COUNTS_T)
    msk = jnp.asarray(_MASK_NEG)

    spec = pl.BlockSpec((L, 2 * E), lambda h: (0, h))
    spec_c = pl.BlockSpec((L, L), lambda h: (0, 0))
    out = pl.pallas_call(
        _body,
        grid=(H // 2,),
        in_specs=[spec, spec, spec, spec_c, spec_c],
        out_specs=spec,
        out_shape=jax.ShapeDtypeStruct((L, H * E), jnp.float32),
        compiler_params=pltpu.CompilerParams(
            dimension_semantics=("parallel",)),
    )(q2, k2, v2, counts_t, msk)
    return out.reshape(B, Lq, Hn, En)


# speedup vs baseline: 1.2557x; 1.1117x over previous
import math

import numpy as np
import jax
import jax.numpy as jnp
from jax.experimental import pallas as pl
from jax.experimental.pallas import tpu as pltpu

L = 2048
H = 16
E = 64
SAMPLE_K = 40
N_TOP = 40
SCALE = 1.0 / math.sqrt(E)
KT = 1024
BT = 256


def _threefry2x32(k0, k1, x0, x1):
    def rotl(x, r):
        return ((x << np.uint32(r)) | (x >> np.uint32(32 - r))).astype(np.uint32)

    R = [13, 15, 26, 6, 17, 29, 16, 24]
    ks0, ks1 = np.uint32(k0), np.uint32(k1)
    ks2 = np.uint32(ks0 ^ ks1 ^ np.uint32(0x1BD11BDA))
    x0 = (x0 + ks0).astype(np.uint32)
    x1 = (x1 + ks1).astype(np.uint32)
    inject = [(ks1, ks2), (ks2, ks0), (ks0, ks1), (ks1, ks2), (ks2, ks0)]
    for g in range(5):
        for r in (R[0:4] if g % 2 == 0 else R[4:8]):
            x0 = (x0 + x1).astype(np.uint32)
            x1 = (rotl(x1, r) ^ x0).astype(np.uint32)
        a, b = inject[g]
        x0 = (x0 + a).astype(np.uint32)
        x1 = (x1 + b + np.uint32(g + 1)).astype(np.uint32)
    return x0, x1


def _sample_counts_t() -> np.ndarray:
    a, b = _threefry2x32(0, 42, np.zeros(2, np.uint32),
                         np.arange(2, dtype=np.uint32))
    k2 = (a[1], b[1])
    i = np.arange(L * SAMPLE_K, dtype=np.uint64)
    hi = (i >> np.uint64(32)).astype(np.uint32)
    lo = (i & np.uint64(0xFFFFFFFF)).astype(np.uint32)
    y0, y1 = _threefry2x32(k2[0], k2[1], hi, lo)
    idx = ((y0 ^ y1) % np.uint32(L)).astype(np.int32).reshape(L, SAMPLE_K)
    cnt = np.zeros((L, L), dtype=np.int8)
    np.add.at(cnt, (idx, np.arange(L)[:, None]), 1)
    return cnt


_COUNTS_T = _sample_counts_t()
_MASK_NEG = np.where(_COUNTS_T > 0, np.float32(0.0),
                     np.float32(-1e30)).astype(np.float32)


def _split(x):
    hi = x.astype(jnp.bfloat16)
    lo = (x - hi.astype(jnp.float32)).astype(jnp.bfloat16)
    return hi, lo


def _mm(a, b, dims):
    return jax.lax.dot_general(a, b, (dims, ((), ())),
                               preferred_element_type=jnp.float32)


def _mm3(a, b, dims):
    ah, al = _split(a)
    bh, bl = _split(b)
    return _mm(ah, bh, dims) + (_mm(ah, bl, dims) + _mm(al, bh, dims))


def _cumsum_block(v):
    ri = jax.lax.broadcasted_iota(jnp.int32, (BT, BT), 0)
    ci = jax.lax.broadcasted_iota(jnp.int32, (BT, BT), 1)
    trib = (ri >= ci).astype(jnp.bfloat16)
    vh, vl = _split(v)
    prefix = jnp.zeros((1, v.shape[1]), jnp.float32)
    blocks = []
    for b in range(L // BT):
        sl = slice(b * BT, (b + 1) * BT)
        cb = (_mm(trib, vh[sl], ((1,), (0,))) +
              _mm(trib, vl[sl], ((1,), (0,))) + prefix)
        blocks.append(cb)
        prefix = cb[BT - 1:BT, :]
    return jnp.concatenate(blocks, axis=0)


def _sparsity(q, k, c_ref, m_ref):
    qb = q.astype(jnp.bfloat16)
    kb = k.astype(jnp.bfloat16)
    run_max = jnp.full((1, L), -jnp.inf, dtype=jnp.float32)
    run_sum = jnp.zeros((1, L), dtype=jnp.float32)
    for t in range(L // KT):
        ktile = kb[t * KT:(t + 1) * KT, :]
        st = _mm(ktile, qb, ((1,), (1,)))
        cf = c_ref[t * KT:(t + 1) * KT, :].astype(jnp.float32)
        run_sum = run_sum + jnp.sum(st * cf, axis=0, keepdims=True)
        masked = st + m_ref[t * KT:(t + 1) * KT, :]
        run_max = jnp.maximum(run_max, jnp.max(masked, axis=0, keepdims=True))
    return run_max - run_sum * (1.0 / SAMPLE_K), qb, kb


def _select2(sp0, sp1):
    iota_row = jax.lax.broadcasted_iota(jnp.int32, (1, L), 1)
    iota16c = jax.lax.broadcasted_iota(jnp.int32, (16, 1), 0)
    bias = jnp.int32(-2 ** 31)

    def sortable(sp):
        bits = jax.lax.bitcast_convert_type(sp, jnp.int32)
        return jnp.where(bits < 0, bits ^ jnp.int32(0x7FFFFFFF), bits)

    u0, u1 = sortable(sp0), sortable(sp1)
    ub = jnp.concatenate([jnp.broadcast_to(u0, (16, L)),
                          jnp.broadcast_to(u1, (16, L))], axis=0)

    tp0 = jnp.zeros((1, 1), jnp.int32)
    tp1 = jnp.zeros((1, 1), jnp.int32)
    for b in range(28, -1, -4):
        th = jnp.concatenate([tp0 | (iota16c << b),
                              tp1 | (iota16c << b)], axis=0)
        cnt = jnp.sum(jnp.where(ub >= (th ^ bias), 1, 0),
                      axis=1, keepdims=True)
        ok = jnp.where(cnt >= N_TOP, 1, 0)
        nib0 = jnp.sum(ok[0:16], axis=0, keepdims=True) - 1
        nib1 = jnp.sum(ok[16:32], axis=0, keepdims=True) - 1
        tp0 = tp0 | (nib0 << b)
        tp1 = tp1 | (nib1 << b)

    def head_eq(u, tp):
        thr = tp ^ bias
        gt = u > thr
        eq = jnp.where(u == thr, 1, 0)
        need = N_TOP - jnp.sum(jnp.where(gt, 1, 0), keepdims=True)
        return gt, eq, need

    gt0, eq0, need0 = head_eq(u0, tp0)
    gt1, eq1, need1 = head_eq(u1, tp1)
    eqb = jnp.concatenate([jnp.broadcast_to(eq0, (16, L)),
                           jnp.broadcast_to(eq1, (16, L))], axis=0)

    ip0 = jnp.zeros((1, 1), jnp.int32)
    ip1 = jnp.zeros((1, 1), jnp.int32)
    for b in (8, 4, 0):
        th_hi = jnp.concatenate([ip0 + ((iota16c + 1) << b) - 1,
                                 ip1 + ((iota16c + 1) << b) - 1], axis=0)
        cnt = jnp.sum(jnp.where(iota_row <= th_hi, eqb, 0),
                      axis=1, keepdims=True)
        nib0 = jnp.sum(jnp.where(cnt[0:16] < need0, 1, 0),
                       axis=0, keepdims=True)
        nib1 = jnp.sum(jnp.where(cnt[16:32] < need1, 1, 0),
                       axis=0, keepdims=True)
        ip0 = ip0 + (nib0 << b)
        ip1 = ip1 + (nib1 << b)
    memb0 = gt0 | ((eq0 > 0) & (iota_row <= ip0))
    memb1 = gt1 | ((eq1 > 0) & (iota_row <= ip1))

    memb0_f = jnp.where(memb0, 1.0, 0.0)
    memb1_f = jnp.where(memb1, 1.0, 0.0)
    m2 = jnp.concatenate(
        [memb0_f[:, i * 128:(i + 1) * 128] for i in range(L // 128)] +
        [memb1_f[:, i * 128:(i + 1) * 128] for i in range(L // 128)], axis=0)
    m2b = m2.astype(jnp.bfloat16)
    ur = jax.lax.broadcasted_iota(jnp.int32, (128, 128), 0)
    uc = jax.lax.broadcasted_iota(jnp.int32, (128, 128), 1)
    triu_b = (ur <= uc).astype(jnp.bfloat16)
    within = _mm(m2b, triu_b, ((1,), (0,)))
    rowtot_b = within[:, 127:128].astype(jnp.bfloat16)
    r32 = jax.lax.broadcasted_iota(jnp.int32, (32, 32), 0)
    c32 = jax.lax.broadcasted_iota(jnp.int32, (32, 32), 1)
    t32_b = ((r32 > c32) &
             ((r32 >> 4) == (c32 >> 4))).astype(jnp.bfloat16)
    offs = _mm(t32_b, rowtot_b, ((1,), (0,)))
    rank = within + offs - m2

    col40f = jax.lax.broadcasted_iota(
        jnp.int32, (N_TOP, 1), 0).astype(jnp.float32)
    iota_rowf = iota_row.astype(jnp.float32)

    def head_sel(rank_rows, memb):
        rank_row = jnp.concatenate(
            [rank_rows[s:s + 1, :] for s in range(L // 128)], axis=1)
        p_t = jnp.where((rank_row == col40f) & memb, 1.0, 0.0)
        pb = p_t.astype(jnp.bfloat16)
        ti_col = jnp.sum(p_t * iota_rowf, axis=1,
                         keepdims=True).astype(jnp.int32)
        return pb, ti_col

    pb0, ti0 = head_sel(rank[0:16], memb0)
    pb1, ti1 = head_sel(rank[16:32], memb1)
    return pb0, ti0, pb1, ti1


def _attend(v, ctx, qb, kb, pb, ti_col):
    q_top = _mm(pb, qb, ((1,), (0,)))
    scores = _mm(q_top.astype(jnp.bfloat16), kb, ((1,), (1,))) * SCALE
    key_pos = jax.lax.broadcasted_iota(jnp.int32, (N_TOP, L), 1)
    scores = jnp.where(key_pos > ti_col, -jnp.inf, scores)
    smax = jnp.max(scores, axis=1, keepdims=True)
    ex = jnp.exp(scores - smax)
    attn = ex / jnp.sum(ex, axis=1, keepdims=True)
    updates = _mm(attn.astype(jnp.bfloat16), v.astype(jnp.bfloat16),
                  ((1,), (0,)))
    scattered = _mm(pb, updates.astype(jnp.bfloat16), ((0,), (0,)))
    ones40_b = jnp.ones((N_TOP, 1), jnp.bfloat16)
    is_top = _mm(pb, ones40_b, ((0,), (0,))) > 0.0
    return jnp.where(is_top, scattered, ctx)


def _body(q_ref, k_ref, v_ref, c_ref, m_ref, o_ref):
    ctx2 = _cumsum_block(v_ref[...])
    s0, s1 = slice(0, E), slice(E, 2 * E)
    sp0, qb0, kb0 = _sparsity(q_ref[:, s0], k_ref[:, s0], c_ref, m_ref)
    sp1, qb1, kb1 = _sparsity(q_ref[:, s1], k_ref[:, s1], c_ref, m_ref)
    pb0, ti0, pb1, ti1 = _select2(sp0, sp1)
    o_ref[:, s0] = _attend(v_ref[:, s0], ctx2[:, s0], qb0, kb0, pb0, ti0)
    o_ref[:, s1] = _attend(v_ref[:, s1], ctx2[:, s1], qb1, kb1, pb1, ti1)


def kernel(queries, keys, values):
    B, Lq, Hn, En = queries.shape
    q2 = queries.reshape(L, H * E)
    k2 = keys.reshape(L, H * E)
    v2 = values.reshape(L, H * E)
    counts_t = jnp.asarray(_COUNTS_T)
    msk = jnp.asarray(_MASK_NEG)

    spec = pl.BlockSpec((L, 2 * E), lambda h: (0, h))
    spec_c = pl.BlockSpec((L, L), lambda h: (0, 0))
    out = pl.pallas_call(
        _body,
        grid=(H // 2,),
        in_specs=[spec, spec, spec, spec_c, spec_c],
        out_specs=spec,
        out_shape=jax.ShapeDtypeStruct((L, H * E), jnp.float32),
        compiler_params=pltpu.CompilerParams(
            dimension_semantics=("parallel",)),
    )(q2, k2, v2, counts_t, msk)
    return out.reshape(B, Lq, Hn, En)
